# baseline (speedup 1.0000x reference)
import functools

import jax
import jax.numpy as jnp
from jax import lax
from jax.experimental import pallas as pl
from jax.experimental.pallas import tpu as pltpu
from jax.experimental.pallas import tpu_sc as plsc

TOPK = 13
EPS = 1e-09
IOU_EPS = 1e-09


def _rot_minmax(cx, cy, w, h, r):
    cos_r = jnp.cos(r)
    sin_r = jnp.sin(r)
    dx = w * 0.5 * cos_r
    dy = h * 0.5 * sin_r
    xm, xp = cx - dx, cx + dx
    ym, yp = cy - dy, cy + dy
    xs = (xm, xp, xp, xm)
    ys = (ym, ym, yp, yp)
    xr = [cx + (xc - cx) * cos_r - (yc - cy) * sin_r for xc, yc in zip(xs, ys)]
    yr = [cy + (xc - cx) * sin_r + (yc - cy) * cos_r for xc, yc in zip(xs, ys)]
    minx = jnp.minimum(jnp.minimum(xr[0], xr[1]), jnp.minimum(xr[2], xr[3]))
    maxx = jnp.maximum(jnp.maximum(xr[0], xr[1]), jnp.maximum(xr[2], xr[3]))
    miny = jnp.minimum(jnp.minimum(yr[0], yr[1]), jnp.minimum(yr[2], yr[3]))
    maxy = jnp.maximum(jnp.maximum(yr[0], yr[1]), jnp.maximum(yr[2], yr[3]))
    return minx, maxx, miny, maxy


def _tc_body(ps_ref, rb_ref, ap_ref, gtl_ref, gtb_ref, gtc_ref, pad_ref,
             bg_ref, out_ref):
    f32 = jnp.float32
    psT = ps_ref[0]
    rbT = rb_ref[0]
    apT = ap_ref[...]
    gtb = gtb_ref[0]
    gtl = gtl_ref[0]
    gtc = gtc_ref[0]
    pad = pad_ref[0]
    C, L = psT.shape
    n = gtb.shape[0]

    pcx, pcy = rbT[0:1, :], rbT[1:2, :]
    pw, ph, pr = rbT[2:3, :], rbT[3:4, :], rbT[4:5, :]
    px, py = apT[0:1, :], apT[1:2, :]

    gcx, gcy = gtb[:, 0:1], gtb[:, 1:2]
    gw, gh, gr = gtb[:, 2:3], gtb[:, 3:4], gtb[:, 4:5]

    g_minx, g_maxx, g_miny, g_maxy = _rot_minmax(gcx, gcy, gw, gh, gr)
    p_minx, p_maxx, p_miny, p_maxy = _rot_minmax(pcx, pcy, pw, ph, pr)

    iw = jnp.clip(jnp.minimum(g_maxx, p_maxx) - jnp.maximum(g_minx, p_minx),
                  0.0, None)
    ih = jnp.clip(jnp.minimum(g_maxy, p_maxy) - jnp.maximum(g_miny, p_miny),
                  0.0, None)
    inter = iw * ih
    union = gw * gh + pw * ph - inter
    iou = jnp.clip(inter / (union + IOU_EPS), 0.0, 1.0)

    cls_oh = (gtl == lax.broadcasted_iota(jnp.int32, (n, C), 1)).astype(f32)
    cls = lax.dot_general(cls_oh, psT, (((1,), (0,)), ((), ())),
                          precision=lax.Precision.HIGHEST,
                          preferred_element_type=f32)

    iou2 = iou * iou
    iou4 = iou2 * iou2
    metric = cls * (iou4 * iou2)

    cosg = jnp.cos(gr)
    sing = jnp.sin(gr)
    dxm = px - gcx
    dym = py - gcy
    lx = dxm * cosg + dym * sing
    ly = -dxm * sing + dym * cosg
    inside = ((jnp.abs(lx) <= gw * 0.5) & (jnp.abs(ly) <= gh * 0.5)).astype(f32)

    iota_l = lax.broadcasted_iota(jnp.int32, (n, L), 1)
    cur = metric * inside
    tk = jnp.zeros((n, L), f32)
    for _ in range(TOPK):
        m = jnp.max(cur, axis=1, keepdims=True)
        idx = jnp.min(jnp.where(cur == m, iota_l, L), axis=1, keepdims=True)
        sel = iota_l == idx
        tk = jnp.where(sel, 1.0, tk)
        cur = jnp.where(sel, -1.0, cur)

    mask_pos = tk * pad * inside
    mps = jnp.sum(mask_pos, axis=0, keepdims=True)
    multiple = mps > 1.0

    iota_g = lax.broadcasted_iota(jnp.int32, (n, L), 0)
    mg = jnp.max(iou, axis=0, keepdims=True)
    gidx = jnp.min(jnp.where(iou == mg, iota_g, n), axis=0, keepdims=True)
    onehot_max = (iota_g == gidx).astype(f32)
    fm = jnp.where(multiple, onehot_max, mask_pos)

    any_pos = jnp.sum(fm, axis=0, keepdims=True) > 0.0
    agi = jnp.sum(fm * iota_g.astype(f32), axis=0, keepdims=True)

    amr = metric * fm
    mm = jnp.max(amr, axis=1, keepdims=True)
    mi = jnp.max(iou * fm, axis=1, keepdims=True)
    amn = amr / (mm + EPS) * mi
    amv = jnp.max(amn, axis=0, keepdims=True)

    bgf = bg_ref[0, 0].astype(f32)
    lab = jnp.where(any_pos, jnp.sum(fm * gtl.astype(f32), axis=0,
                                     keepdims=True), bgf)
    crwf = gtc.astype(f32)
    crw = jnp.where(any_pos, jnp.sum(fm * crwf, axis=0, keepdims=True),
                    crwf[0, 0])

    out_ref[0, 0:1, :] = lab
    out_ref[0, 1:2, :] = amv
    out_ref[0, 2:3, :] = agi
    out_ref[0, 3:4, :] = crw


def _run_tc(psT, rbT, apT, gt_labels, gt_bboxes, gt_crowd, pad_gt_mask,
            bg_arr, B, L, C, n):
    return pl.pallas_call(
        _tc_body,
        grid=(B,),
        in_specs=[
            pl.BlockSpec((1, C, L), lambda b: (b, 0, 0)),
            pl.BlockSpec((1, 5, L), lambda b: (b, 0, 0)),
            pl.BlockSpec((2, L), lambda b: (0, 0)),
            pl.BlockSpec((1, n, 1), lambda b: (b, 0, 0)),
            pl.BlockSpec((1, n, 5), lambda b: (b, 0, 0)),
            pl.BlockSpec((1, n, 1), lambda b: (b, 0, 0)),
            pl.BlockSpec((1, n, 1), lambda b: (b, 0, 0)),
            pl.BlockSpec((1, 1), lambda b: (0, 0)),
        ],
        out_specs=pl.BlockSpec((1, 8, L), lambda b: (b, 0, 0)),
        out_shape=jax.ShapeDtypeStruct((B, 8, L), jnp.float32),
    )(psT, rbT, apT, gt_labels, gt_bboxes, gt_crowd, pad_gt_mask, bg_arr)


def _make_sc_assemble(B, L, C):
    NW = 32
    W = 640
    W_LAST = L - (NW - 1) * W
    mesh = plsc.VectorSubcoreMesh(core_axis_name="c", subcore_axis_name="s")

    @functools.partial(
        pl.kernel,
        mesh=mesh,
        compiler_params=pltpu.CompilerParams(needs_layout_passes=False,
                                             use_tc_tiling_on_sc=False),
        out_type=(
            jax.ShapeDtypeStruct((B, L), jnp.int32),
            jax.ShapeDtypeStruct((B, L * 5), jnp.float32),
            jax.ShapeDtypeStruct((B, L * C), jnp.float32),
            jax.ShapeDtypeStruct((B, L), jnp.int32),
            jax.ShapeDtypeStruct((B, L), jnp.int32),
        ),
        scratch_types=[
            pltpu.VMEM((W,), jnp.float32),
            pltpu.VMEM((W,), jnp.float32),
            pltpu.VMEM((W,), jnp.float32),
            pltpu.VMEM((W,), jnp.float32),
            pltpu.VMEM((320,), jnp.float32),
            pltpu.VMEM((W,), jnp.int32),
            pltpu.VMEM((W,), jnp.int32),
            pltpu.VMEM((W,), jnp.int32),
            pltpu.VMEM((W * C,), jnp.float32),
            pltpu.VMEM((W * 5,), jnp.float32),
        ],
    )
    def sc_assemble(compact, gtb_flat, lab_o, rbx_o, sc_o, agi_o, crw_o,
                    lab_v, amv_v, agi_v, crw_v, gtb_v,
                    lab_i, agi_i, crw_i, sc_buf, bx_buf):
        wid = lax.axis_index("s") * 2 + lax.axis_index("c")
        lane = jnp.arange(16, dtype=jnp.int32)

        def span(base, nvec, wspan):
            def per_batch(b, _):
                pltpu.sync_copy(compact.at[b, 0, pl.ds(base, wspan)],
                                lab_v.at[pl.ds(0, wspan)])
                pltpu.sync_copy(compact.at[b, 1, pl.ds(base, wspan)],
                                amv_v.at[pl.ds(0, wspan)])
                pltpu.sync_copy(compact.at[b, 2, pl.ds(base, wspan)],
                                agi_v.at[pl.ds(0, wspan)])
                pltpu.sync_copy(compact.at[b, 3, pl.ds(base, wspan)],
                                crw_v.at[pl.ds(0, wspan)])
                pltpu.sync_copy(gtb_flat.at[b], gtb_v)

                def per_vec(j, _):
                    o = j * 16
                    labf = lab_v[pl.ds(o, 16)]
                    amv16 = amv_v[pl.ds(o, 16)]
                    lab16 = labf.astype(jnp.int32)
                    agi16 = agi_v[pl.ds(o, 16)].astype(jnp.int32)
                    crw16 = crw_v[pl.ds(o, 16)].astype(jnp.int32)
                    lab_i[pl.ds(o, 16)] = lab16
                    agi_i[pl.ds(o, 16)] = agi16
                    crw_i[pl.ds(o, 16)] = crw16
                    lidx = o + lane
                    gbase = agi16 * 5
                    for c in range(C):
                        vals = jnp.where(lab16 == c, amv16, 0.0)
                        plsc.store_scatter(sc_buf, [lidx * C + c], vals)
                    for c in range(5):
                        g = plsc.load_gather(gtb_v, [gbase + c])
                        plsc.store_scatter(bx_buf, [lidx * 5 + c], g)
                    return 0

                lax.fori_loop(0, nvec, per_vec, 0)

                pltpu.sync_copy(lab_i.at[pl.ds(0, wspan)],
                                lab_o.at[b, pl.ds(base, wspan)])
                pltpu.sync_copy(agi_i.at[pl.ds(0, wspan)],
                                agi_o.at[b, pl.ds(base, wspan)])
                pltpu.sync_copy(crw_i.at[pl.ds(0, wspan)],
                                crw_o.at[b, pl.ds(base, wspan)])
                pltpu.sync_copy(sc_buf.at[pl.ds(0, wspan * C)],
                                sc_o.at[b, pl.ds(base * C, wspan * C)])
                pltpu.sync_copy(bx_buf.at[pl.ds(0, wspan * 5)],
                                rbx_o.at[b, pl.ds(base * 5, wspan * 5)])
                return 0

            lax.fori_loop(0, B, per_batch, 0)

        @pl.when(wid < NW - 1)
        def _():
            span(wid * W, W // 16, W)

        @pl.when(wid == NW - 1)
        def _():
            span((NW - 1) * W, W_LAST // 16, W_LAST)

    return sc_assemble


def kernel(pred_scores, pred_rboxes, anchor_points, gt_labels, gt_bboxes,
           gt_poses, gt_crowd, pad_gt_mask, bg_index):
    B, L, C = pred_scores.shape
    n = gt_bboxes.shape[1]
    bg_arr = jnp.reshape(jnp.asarray(bg_index, jnp.int32), (1, 1))
    psT = jnp.transpose(pred_scores, (0, 2, 1))
    rbT = jnp.transpose(pred_rboxes, (0, 2, 1))
    apT = jnp.transpose(anchor_points, (1, 0))

    compact = _run_tc(psT, rbT, apT, gt_labels, gt_bboxes, gt_crowd,
                      pad_gt_mask, bg_arr, B, L, C, n)

    gtb_flat = jnp.pad(gt_bboxes.reshape(B, n * 5), ((0, 0), (0, 320 - n * 5)))
    labels, rboxes, scores, agi, crw = _make_sc_assemble(B, L, C)(
        compact, gtb_flat)
    return (labels, rboxes.reshape(B, L, 5), scores.reshape(B, L, C), agi,
            crw.astype(bool))


# speedup vs baseline: 1.0125x; 1.0125x over previous
import functools

import jax
import jax.numpy as jnp
from jax import lax
from jax.experimental import pallas as pl
from jax.experimental.pallas import tpu as pltpu
from jax.experimental.pallas import tpu_sc as plsc

TOPK = 13
EPS = 1e-09
IOU_EPS = 1e-09


def _rot_minmax(cx, cy, w, h, r):
    cos_r = jnp.cos(r)
    sin_r = jnp.sin(r)
    dx = w * 0.5 * cos_r
    dy = h * 0.5 * sin_r
    xm, xp = cx - dx, cx + dx
    ym, yp = cy - dy, cy + dy
    xs = (xm, xp, xp, xm)
    ys = (ym, ym, yp, yp)
    xr = [cx + (xc - cx) * cos_r - (yc - cy) * sin_r for xc, yc in zip(xs, ys)]
    yr = [cy + (xc - cx) * sin_r + (yc - cy) * cos_r for xc, yc in zip(xs, ys)]
    minx = jnp.minimum(jnp.minimum(xr[0], xr[1]), jnp.minimum(xr[2], xr[3]))
    maxx = jnp.maximum(jnp.maximum(xr[0], xr[1]), jnp.maximum(xr[2], xr[3]))
    miny = jnp.minimum(jnp.minimum(yr[0], yr[1]), jnp.minimum(yr[2], yr[3]))
    maxy = jnp.maximum(jnp.maximum(yr[0], yr[1]), jnp.maximum(yr[2], yr[3]))
    return minx, maxx, miny, maxy


def _tc_body(ps_ref, rb_ref, ap_ref, gtl_ref, gtb_ref, gtc_ref, pad_ref,
             bg_ref, out_ref):
    f32 = jnp.float32
    psT = ps_ref[0]
    rbT = rb_ref[0]
    apT = ap_ref[...]
    gtb = gtb_ref[0]
    gtl = gtl_ref[0]
    gtc = gtc_ref[0]
    pad = pad_ref[0]
    C, L = psT.shape
    n = gtb.shape[0]

    pcx, pcy = rbT[0:1, :], rbT[1:2, :]
    pw, ph, pr = rbT[2:3, :], rbT[3:4, :], rbT[4:5, :]
    px, py = apT[0:1, :], apT[1:2, :]

    gcx, gcy = gtb[:, 0:1], gtb[:, 1:2]
    gw, gh, gr = gtb[:, 2:3], gtb[:, 3:4], gtb[:, 4:5]

    g_minx, g_maxx, g_miny, g_maxy = _rot_minmax(gcx, gcy, gw, gh, gr)
    p_minx, p_maxx, p_miny, p_maxy = _rot_minmax(pcx, pcy, pw, ph, pr)

    iw = jnp.clip(jnp.minimum(g_maxx, p_maxx) - jnp.maximum(g_minx, p_minx),
                  0.0, None)
    ih = jnp.clip(jnp.minimum(g_maxy, p_maxy) - jnp.maximum(g_miny, p_miny),
                  0.0, None)
    inter = iw * ih
    union = gw * gh + pw * ph - inter
    iou = jnp.clip(inter / (union + IOU_EPS), 0.0, 1.0)

    cls_oh = (gtl == lax.broadcasted_iota(jnp.int32, (n, C), 1)).astype(f32)
    cls = lax.dot_general(cls_oh, psT, (((1,), (0,)), ((), ())),
                          precision=lax.Precision.HIGHEST,
                          preferred_element_type=f32)

    iou2 = iou * iou
    iou4 = iou2 * iou2
    metric = cls * (iou4 * iou2)

    cosg = jnp.cos(gr)
    sing = jnp.sin(gr)
    dxm = px - gcx
    dym = py - gcy
    lx = dxm * cosg + dym * sing
    ly = -dxm * sing + dym * cosg
    inside = ((jnp.abs(lx) <= gw * 0.5) & (jnp.abs(ly) <= gh * 0.5)).astype(f32)

    iota_l = lax.broadcasted_iota(jnp.int32, (n, L), 1)
    cur = metric * inside
    tk = jnp.zeros((n, L), f32)
    for _ in range(TOPK):
        m = jnp.max(cur, axis=1, keepdims=True)
        idx = jnp.min(jnp.where(cur == m, iota_l, L), axis=1, keepdims=True)
        sel = iota_l == idx
        tk = jnp.where(sel, 1.0, tk)
        cur = jnp.where(sel, -1.0, cur)

    mask_pos = tk * pad * inside
    mps = jnp.sum(mask_pos, axis=0, keepdims=True)
    multiple = mps > 1.0

    iota_g = lax.broadcasted_iota(jnp.int32, (n, L), 0)
    mg = jnp.max(iou, axis=0, keepdims=True)
    gidx = jnp.min(jnp.where(iou == mg, iota_g, n), axis=0, keepdims=True)
    onehot_max = (iota_g == gidx).astype(f32)
    fm = jnp.where(multiple, onehot_max, mask_pos)

    any_pos = jnp.sum(fm, axis=0, keepdims=True) > 0.0
    agi = jnp.sum(fm * iota_g.astype(f32), axis=0, keepdims=True)

    amr = metric * fm
    mm = jnp.max(amr, axis=1, keepdims=True)
    mi = jnp.max(iou * fm, axis=1, keepdims=True)
    amn = amr / (mm + EPS) * mi
    amv = jnp.max(amn, axis=0, keepdims=True)

    bgf = bg_ref[0, 0].astype(f32)
    lab = jnp.where(any_pos, jnp.sum(fm * gtl.astype(f32), axis=0,
                                     keepdims=True), bgf)
    crwf = gtc.astype(f32)
    crw = jnp.where(any_pos, jnp.sum(fm * crwf, axis=0, keepdims=True),
                    crwf[0, 0])

    out_ref[0, 0:1, :] = lab
    out_ref[0, 1:2, :] = amv
    out_ref[0, 2:3, :] = agi
    out_ref[0, 3:4, :] = crw


def _run_tc(psT, rbT, apT, gt_labels, gt_bboxes, gt_crowd, pad_gt_mask,
            bg_arr, B, L, C, n):
    return pl.pallas_call(
        _tc_body,
        grid=(B,),
        in_specs=[
            pl.BlockSpec((1, C, L), lambda b: (b, 0, 0)),
            pl.BlockSpec((1, 5, L), lambda b: (b, 0, 0)),
            pl.BlockSpec((2, L), lambda b: (0, 0)),
            pl.BlockSpec((1, n, 1), lambda b: (b, 0, 0)),
            pl.BlockSpec((1, n, 5), lambda b: (b, 0, 0)),
            pl.BlockSpec((1, n, 1), lambda b: (b, 0, 0)),
            pl.BlockSpec((1, n, 1), lambda b: (b, 0, 0)),
            pl.BlockSpec((1, 1), lambda b: (0, 0)),
        ],
        out_specs=pl.BlockSpec((1, 8, L), lambda b: (b, 0, 0)),
        out_shape=jax.ShapeDtypeStruct((B, 8, L), jnp.float32),
    )(psT, rbT, apT, gt_labels, gt_bboxes, gt_crowd, pad_gt_mask, bg_arr)


def _make_sc_assemble(B, L, C):
    NW = 32
    W = 640
    W_LAST = L - (NW - 1) * W
    mesh = plsc.VectorSubcoreMesh(core_axis_name="c", subcore_axis_name="s")

    @functools.partial(
        pl.kernel,
        mesh=mesh,
        compiler_params=pltpu.CompilerParams(needs_layout_passes=False,
                                             use_tc_tiling_on_sc=False),
        out_type=(
            jax.ShapeDtypeStruct((B, 3, L), jnp.int32),
            jax.ShapeDtypeStruct((B, L * 5), jnp.float32),
            jax.ShapeDtypeStruct((B, L * C), jnp.float32),
        ),
        scratch_types=[
            pltpu.VMEM((4, W), jnp.float32),
            pltpu.VMEM((B, 320), jnp.float32),
            pltpu.VMEM((3, W), jnp.int32),
            pltpu.VMEM((W * C,), jnp.float32),
            pltpu.VMEM((W * 5,), jnp.float32),
        ],
    )
    def sc_assemble(compact, gtb_flat, int_o, rbx_o, sc_o,
                    cin, gtb_v, iout, sc_buf, bx_buf):
        wid = lax.axis_index("s") * 2 + lax.axis_index("c")
        lane = jnp.arange(16, dtype=jnp.int32)

        def span(base, nvec, wspan):
            pltpu.sync_copy(gtb_flat, gtb_v)

            def per_batch(b, _):
                pltpu.sync_copy(compact.at[b, 0:4, pl.ds(base, wspan)],
                                cin.at[:, pl.ds(0, wspan)])
                bvec = jnp.full((16,), b, jnp.int32)

                def per_vec(j, _):
                    o = j * 16
                    labf = cin[0, pl.ds(o, 16)]
                    amv16 = cin[1, pl.ds(o, 16)]
                    lab16 = labf.astype(jnp.int32)
                    agi16 = cin[2, pl.ds(o, 16)].astype(jnp.int32)
                    crw16 = cin[3, pl.ds(o, 16)].astype(jnp.int32)
                    iout[0, pl.ds(o, 16)] = lab16
                    iout[1, pl.ds(o, 16)] = agi16
                    iout[2, pl.ds(o, 16)] = crw16
                    lidx = o + lane
                    gbase = agi16 * 5
                    for c in range(C):
                        vals = jnp.where(lab16 == c, amv16, 0.0)
                        plsc.store_scatter(sc_buf, [lidx * C + c], vals)
                    for c in range(5):
                        g = plsc.load_gather(gtb_v, [bvec, gbase + c])
                        plsc.store_scatter(bx_buf, [lidx * 5 + c], g)
                    return 0

                lax.fori_loop(0, nvec, per_vec, 0)

                pltpu.sync_copy(iout.at[:, pl.ds(0, wspan)],
                                int_o.at[b, :, pl.ds(base, wspan)])
                pltpu.sync_copy(sc_buf.at[pl.ds(0, wspan * C)],
                                sc_o.at[b, pl.ds(base * C, wspan * C)])
                pltpu.sync_copy(bx_buf.at[pl.ds(0, wspan * 5)],
                                rbx_o.at[b, pl.ds(base * 5, wspan * 5)])
                return 0

            lax.fori_loop(0, B, per_batch, 0)

        @pl.when(wid < NW - 1)
        def _():
            span(wid * W, W // 16, W)

        @pl.when(wid == NW - 1)
        def _():
            span((NW - 1) * W, W_LAST // 16, W_LAST)

    return sc_assemble


def kernel(pred_scores, pred_rboxes, anchor_points, gt_labels, gt_bboxes,
           gt_poses, gt_crowd, pad_gt_mask, bg_index):
    B, L, C = pred_scores.shape
    n = gt_bboxes.shape[1]
    bg_arr = jnp.reshape(jnp.asarray(bg_index, jnp.int32), (1, 1))
    psT = jnp.transpose(pred_scores, (0, 2, 1))
    rbT = jnp.transpose(pred_rboxes, (0, 2, 1))
    apT = jnp.transpose(anchor_points, (1, 0))

    compact = _run_tc(psT, rbT, apT, gt_labels, gt_bboxes, gt_crowd,
                      pad_gt_mask, bg_arr, B, L, C, n)

    gtb_flat = jnp.pad(gt_bboxes.reshape(B, n * 5), ((0, 0), (0, 320 - n * 5)))
    ints, rboxes, scores = _make_sc_assemble(B, L, C)(compact, gtb_flat)
    return (ints[:, 0], rboxes.reshape(B, L, 5), scores.reshape(B, L, C),
            ints[:, 1], ints[:, 2].astype(bool))
